# Initial kernel scaffold; baseline (speedup 1.0000x reference)
import functools

import jax
import jax.numpy as jnp
from jax import lax
from jax.experimental import pallas as pl
from jax.experimental.pallas import tpu as pltpu
from jax.experimental.pallas import tpu_sc as plsc

N = 100000
E = 1600000
C = 32
NC = 2
NS = 16
NT = NC * NS
NPT = 3128
NPAD = NT * NPT
K = 2000
G = 64
CH = 6256
SR = 100
S0R = K // SR
E16 = E // NS
MEID = K + 96

_mesh = plsc.VectorSubcoreMesh(
    core_axis_name="c", subcore_axis_name="s", num_cores=NC, num_subcores=NS)


@functools.partial(
    pl.kernel,
    out_type=jax.ShapeDtypeStruct((NPAD * C,), jnp.float32),
    mesh=_mesh,
    scratch_types=[
        pltpu.VMEM((NPT * C,), jnp.float32),
        pltpu.VMEM((NPT,), jnp.float32),
        pltpu.VMEM((K,), jnp.int32),
        pltpu.VMEM((MEID,), jnp.int32),
        pltpu.VMEM((G, 4), jnp.int32),
        pltpu.VMEM((G,), jnp.int32),
        pltpu.VMEM((G,), jnp.int32),
        pltpu.VMEM((G,), jnp.float32),
        pltpu.VMEM((G,), jnp.int32),
        pltpu.VMEM((G, C), jnp.float32),
        pltpu.VMEM((G, C), jnp.float32),
        pltpu.VMEM((CH,), jnp.float32),
        pltpu.VMEM((S0R, SR), jnp.int32),
        pltpu.VMEM((S0R, SR), jnp.float32),
        pltpu.VMEM_SHARED((NPAD,), jnp.float32),
        pltpu.SemaphoreType.DMA,
        pltpu.SemaphoreType.DMA,
    ],
)
def _eikonal_sc(ytr, srca, src2, w2, reca, maskp, zerosf, out,
                acc, deg_own, src_buf, meid, rec_buf, sidx, didx, rbuf,
                slbuf, ysrc, ydst, mbuf, s0src, s0w, deg_sh, sem1, sem2):
    cid = lax.axis_index("c")
    sid = lax.axis_index("s")
    wid = cid * NS + sid
    lo = wid * NPT
    hi = lo + NPT
    iota = lax.iota(jnp.int32, 16)

    pltpu.sync_copy(zerosf, acc)

    @pl.when(sid == 0)
    def _zero_deg():
        pltpu.sync_copy(zerosf, deg_sh)

    def _mz(k, carry):
        meid[pl.ds(k * 16, 16)] = jnp.zeros((16,), jnp.int32)
        return carry
    lax.fori_loop(0, MEID // 16, _mz, 0)

    plsc.subcore_barrier()

    def _st0(i, carry):
        roff = sid * (E16 // SR) + i * S0R
        pltpu.sync_copy(src2.at[pl.ds(roff, S0R), :], s0src)
        pltpu.sync_copy(w2.at[pl.ds(roff, S0R), :], s0w)

        def _sc(j, c2):
            pltpu.sync_copy(s0w.at[j], deg_sh.at[s0src.at[j]], add=True)
            return c2
        lax.fori_loop(0, S0R, _sc, 0)
        return carry
    lax.fori_loop(0, E16 // K, _st0, 0)

    plsc.subcore_barrier()
    pltpu.sync_copy(deg_sh.at[pl.ds(lo, NPT)], deg_own)

    def _do_group(q0, cnt):
        pltpu.async_copy(reca.at[meid.at[pl.ds(q0, G)]], rec_buf, sem1).wait()

        def _prep(v, carry):
            rows = iota + v * 16
            srcv = plsc.load_gather(rec_buf, [rows, jnp.zeros((16,), jnp.int32)])
            dstv = plsc.load_gather(rec_buf, [rows, jnp.full((16,), 1, jnp.int32)])
            wbits = plsc.load_gather(rec_buf, [rows, jnp.full((16,), 2, jnp.int32)])
            wv = plsc.bitcast(wbits, jnp.float32)
            slv = srcv - lo
            slv = jnp.minimum(jnp.maximum(slv, 0), NPT - 1)
            degv = plsc.load_gather(deg_own, [slv])
            yb = 0x5F3759DF - lax.shift_right_logical(wbits, 1)
            yv = plsc.bitcast(yb, jnp.float32)
            hw = 0.5 * wv
            yv = yv * (1.5 - hw * yv * yv)
            yv = yv * (1.5 - hw * yv * yv)
            yv = yv * (1.5 - hw * yv * yv)
            rv = wv * yv / degv
            sidx[pl.ds(v * 16, 16)] = srcv
            didx[pl.ds(v * 16, 16)] = dstv
            slbuf[pl.ds(v * 16, 16)] = slv
            rbuf[pl.ds(v * 16, 16)] = rv
            return carry
        lax.fori_loop(0, G // 16, _prep, 0)

        d1 = pltpu.async_copy(ytr.at[sidx], ysrc, sem1)
        d2 = pltpu.async_copy(ytr.at[didx], ydst, sem2)
        d1.wait()
        d2.wait()

        def _rmw(e, carry):
            ef = jnp.full((16,), e, jnp.int32)
            sl = slbuf[e]
            rr = rbuf[e]
            ys0 = plsc.load_gather(ysrc, [ef, iota])
            ys1 = plsc.load_gather(ysrc, [ef, iota + 16])
            yd0 = plsc.load_gather(ydst, [ef, iota])
            yd1 = plsc.load_gather(ydst, [ef, iota + 16])
            v0 = rr * jnp.maximum(ys0 - yd0, 0.0)
            v1 = rr * jnp.maximum(ys1 - yd1, 0.0)
            o = sl * C
            a0 = acc[pl.ds(o, 16)]
            acc[pl.ds(o, 16)] = jnp.maximum(a0, v0)
            a1 = acc[pl.ds(o + 16, 16)]
            acc[pl.ds(o + 16, 16)] = jnp.maximum(a1, v1)
            return carry
        lax.fori_loop(0, cnt, _rmw, 0)

    def _chunk(i, cursor):
        off = i * K
        pltpu.sync_copy(srca.at[pl.ds(off, K)], src_buf)

        def _scan(v, cur):
            sv = src_buf[pl.ds(v * 16, 16)]
            m = (sv >= lo) & (sv < hi)
            eidv = iota + (off + v * 16)
            plsc.store_compressed(meid.at[pl.ds(cur, 16)], eidv, mask=m)
            return cur + jnp.sum(m.astype(jnp.int32))
        cursor = lax.fori_loop(0, K // 16, _scan, cursor)

        nfull = cursor // G

        def _grp(j, carry):
            _do_group(j * G, G)
            return carry
        lax.fori_loop(0, nfull, _grp, 0)

        rem = cursor - nfull * G
        base = nfull * G

        def _mv(k2, carry):
            vals = meid[pl.ds(base + k2 * 16, 16)]
            meid[pl.ds(k2 * 16, 16)] = vals
            return carry
        lax.fori_loop(0, (rem + 15) // 16, _mv, 0)
        return rem
    cursor = lax.fori_loop(0, E // K, _chunk, 0)
    _do_group(0, cursor)

    def _st3(ci2, carry):
        o = ci2 * CH
        pltpu.sync_copy(maskp.at[pl.ds(lo * C + o, CH)], mbuf)

        def _v3(v, c2):
            a = acc[pl.ds(o + v * 16, 16)]
            mk = mbuf[pl.ds(v * 16, 16)]
            mbuf[pl.ds(v * 16, 16)] = (1.0 - a) * mk
            return c2
        lax.fori_loop(0, CH // 16, _v3, 0)
        pltpu.sync_copy(mbuf, out.at[pl.ds(lo * C + o, CH)])
        return carry
    lax.fori_loop(0, (NPT * C) // CH, _st3, 0)


def kernel(t, y, edge_index, edge_attr, mask):
    del t
    src = edge_index[0]
    dst = edge_index[1]
    w = edge_attr
    ytr = y.T
    wbits = lax.bitcast_convert_type(w, jnp.int32)
    rec = jnp.stack([src, dst, wbits, jnp.zeros((E,), jnp.int32)], axis=1)
    maskp = jnp.pad(mask.T, ((0, NPAD - N), (0, 0))).reshape(-1)
    zerosf = jnp.zeros((NPAD,), jnp.float32)
    src2 = src.reshape(E // SR, SR)
    w2 = w.reshape(E // SR, SR)
    outflat = _eikonal_sc(ytr, src, src2, w2, rec, maskp, zerosf)
    return outflat.reshape(NPAD, C)[:N].T


# speedup vs baseline: 3.3318x; 3.3318x over previous
import functools

import jax
import jax.numpy as jnp
from jax import lax
from jax.experimental import pallas as pl
from jax.experimental.pallas import tpu as pltpu
from jax.experimental.pallas import tpu_sc as plsc

N = 100000
E = 1600000
C = 32
NC = 2
NS = 16
NT = NC * NS
NPT = 3136
NPAD = NT * NPT
K = 2000
G = 64
OB = 112
MEID = K + 96

_mesh = plsc.VectorSubcoreMesh(
    core_axis_name="c", subcore_axis_name="s", num_cores=NC, num_subcores=NS)


@functools.partial(
    pl.kernel,
    out_type=jax.ShapeDtypeStruct((NPAD, C), jnp.float32),
    mesh=_mesh,
    compiler_params=pltpu.CompilerParams(needs_layout_passes=False,
                                         use_tc_tiling_on_sc=False),
    scratch_types=[
        pltpu.VMEM((NPT * C,), jnp.float32),
        pltpu.VMEM((NPT + 16,), jnp.float32),
        pltpu.VMEM((K,), jnp.int32),
        pltpu.VMEM((K,), jnp.int32),
        pltpu.VMEM((K,), jnp.float32),
        pltpu.VMEM((MEID,), jnp.int32),
        pltpu.VMEM((MEID,), jnp.int32),
        pltpu.VMEM((MEID,), jnp.float32),
        pltpu.VMEM((G,), jnp.int32),
        pltpu.VMEM((G,), jnp.int32),
        pltpu.VMEM((G + 16,), jnp.float32),
        pltpu.VMEM((G + 16,), jnp.int32),
        pltpu.VMEM((G, C), jnp.float32),
        pltpu.VMEM((G, C), jnp.float32),
        pltpu.VMEM((OB * C,), jnp.float32),
        pltpu.VMEM((OB, C), jnp.float32),
        pltpu.VMEM((OB,), jnp.int32),
        pltpu.VMEM((64,), jnp.int32),
        pltpu.SemaphoreType.DMA,
        pltpu.SemaphoreType.DMA,
    ],
)
def _eikonal_sc(ytr, srca, dsta, wa, maskp, zerosf, out,
                acc, deg, src_buf, dst_buf, w_buf, msrc, mdst, mw,
                sidx, didx, rbuf, slbuf, ysrc, ydst, mbuf, obuf, oidx,
                ptmp, sem1, sem2):
    cid = lax.axis_index("c")
    sid = lax.axis_index("s")
    wid = cid * NS + sid
    lo = wid * NPT
    hi = lo + NPT

    pltpu.sync_copy(zerosf, acc)
    pltpu.sync_copy(zerosf.at[pl.ds(0, NPT + 16)], deg)
    ptmp[pl.ds(0, 16)] = jnp.zeros((16,), jnp.int32)
    ptmp[pl.ds(32, 16)] = jnp.zeros((16,), jnp.int32)

    def _mz(k, carry):
        msrc[pl.ds(k * 16, 16)] = jnp.zeros((16,), jnp.int32)
        mdst[pl.ds(k * 16, 16)] = jnp.zeros((16,), jnp.int32)
        mw[pl.ds(k * 16, 16)] = jnp.full((16,), 1.0, jnp.float32)
        return carry
    lax.fori_loop(0, MEID // 16, _mz, 0)

    def _do_group(q0, cnt):
        def _prep(v, carry):
            iota = lax.iota(jnp.int32, 16)
            rows = iota + v * 16
            srcv = plsc.load_gather(msrc, [rows + q0])
            dstv = plsc.load_gather(mdst, [rows + q0])
            wv0 = plsc.load_gather(mw, [rows + q0])
            lane_ok = jnp.minimum(jnp.maximum(cnt - rows, 0),
                                  1).astype(jnp.float32)
            wv = wv0 * lane_ok + (1.0 - lane_ok)
            slv = srcv - lo
            slv = jnp.minimum(jnp.maximum(slv, 0), NPT - 1)
            wbits = plsc.bitcast(wv, jnp.int32)
            yb = 0x5F3759DF - lax.shift_right_logical(wbits, 1)
            yv = plsc.bitcast(yb, jnp.float32)
            hw = 0.5 * wv
            yv = yv * (1.5 - hw * yv * yv)
            yv = yv * (1.5 - hw * yv * yv)
            yv = yv * (1.5 - hw * yv * yv)
            sidx[pl.ds(v * 16, 16)] = srcv
            didx[pl.ds(v * 16, 16)] = dstv
            slbuf[pl.ds(v * 16, 16)] = slv
            rbuf[pl.ds(v * 16, 16)] = wv * yv
            plsc.addupdate_scatter(deg, [slv], wv * lane_ok)
            return carry
        lax.fori_loop(0, G // 16, _prep, 0)

        d1 = pltpu.async_copy(ytr.at[sidx], ysrc, sem1)
        d2 = pltpu.async_copy(ytr.at[didx], ydst, sem2)
        d1.wait()
        d2.wait()

        def _rmw(e, carry):
            iota = lax.iota(jnp.int32, 16)
            ef = jnp.full((16,), e, jnp.int32)
            sl = slbuf[pl.ds(e, 16)][0]
            rr = rbuf[pl.ds(e, 16)][0]
            ys0 = plsc.load_gather(ysrc, [ef, iota])
            ys1 = plsc.load_gather(ysrc, [ef, iota + 16])
            yd0 = plsc.load_gather(ydst, [ef, iota])
            yd1 = plsc.load_gather(ydst, [ef, iota + 16])
            v0 = rr * jnp.maximum(ys0 - yd0, 0.0)
            v1 = rr * jnp.maximum(ys1 - yd1, 0.0)
            o = sl * C
            a0 = acc[pl.ds(o, 16)]
            acc[pl.ds(o, 16)] = jnp.maximum(a0, v0)
            a1 = acc[pl.ds(o + 16, 16)]
            acc[pl.ds(o + 16, 16)] = jnp.maximum(a1, v1)
            return carry
        lax.fori_loop(0, cnt, _rmw, 0)

    def _chunk(i, cursor):
        off = i * K
        pltpu.sync_copy(srca.at[pl.ds(off, K)], src_buf)
        pltpu.sync_copy(dsta.at[pl.ds(off, K)], dst_buf)
        pltpu.sync_copy(wa.at[pl.ds(off, K)], w_buf)

        def _scan(v, carry):
            sv = src_buf[pl.ds(v * 16, 16)]
            mi = jnp.minimum(jnp.maximum(jnp.minimum(sv - (lo - 1), hi - sv),
                                         0), 1)
            cs = mi
            for d in (1, 2, 4, 8):
                ptmp[pl.ds(16, 16)] = cs
                cs = cs + ptmp[pl.ds(16 - d, 16)]
            curv = ptmp[pl.ds(32, 16)]
            pos = (curv - 1) + cs
            posc = jnp.minimum(jnp.maximum(pos, 0), MEID - 2)
            posf = (MEID - 1) + mi * (posc - (MEID - 1))
            plsc.store_scatter(msrc, [posf], sv)
            plsc.store_scatter(mdst, [posf], dst_buf[pl.ds(v * 16, 16)])
            plsc.store_scatter(mw, [posf], w_buf[pl.ds(v * 16, 16)])
            ptmp[pl.ds(32, 16)] = curv + cs[15]
            return carry
        lax.fori_loop(0, K // 16, _scan, 0)
        cursor = ptmp[pl.ds(32, 16)][0]

        nfull = cursor // G

        def _grp(j, carry):
            _do_group(j * G, G)
            return carry
        lax.fori_loop(0, nfull, _grp, 0)

        rem = cursor - nfull * G
        base = nfull * G

        def _mv(k2, carry):
            msrc[pl.ds(k2 * 16, 16)] = msrc[pl.ds(base + k2 * 16, 16)]
            mdst[pl.ds(k2 * 16, 16)] = mdst[pl.ds(base + k2 * 16, 16)]
            mw[pl.ds(k2 * 16, 16)] = mw[pl.ds(base + k2 * 16, 16)]
            return carry
        lax.fori_loop(0, (rem + 15) // 16, _mv, 0)
        ptmp[pl.ds(32, 16)] = jnp.full((16,), 1, jnp.int32) * rem
        return rem
    cursor = lax.fori_loop(0, E // K, _chunk, 0)
    _do_group(0, cursor)

    def _inv(v, carry):
        d = deg[pl.ds(v * 16, 16)]
        deg[pl.ds(v * 16, 16)] = 1.0 / jnp.maximum(d, 1e-20)
        return carry
    lax.fori_loop(0, NPT // 16, _inv, 0)

    def _st3(ob, carry):
        nb = ob * OB
        pltpu.sync_copy(maskp.at[pl.ds((lo + nb) * C, OB * C)], mbuf)

        def _oi(v, c2):
            iota = lax.iota(jnp.int32, 16)
            oidx[pl.ds(v * 16, 16)] = iota + (lo + nb + v * 16)
            return c2
        lax.fori_loop(0, OB // 16, _oi, 0)

        def _nd(n, c2):
            idv = deg[pl.ds(nb + n, 16)][0]
            a0 = acc[pl.ds((nb + n) * C, 16)]
            mk0 = mbuf[pl.ds(n * C, 16)]
            a1 = acc[pl.ds((nb + n) * C + 16, 16)]
            mk1 = mbuf[pl.ds(n * C + 16, 16)]
            obuf[n, pl.ds(0, 16)] = (1.0 - a0 * idv) * mk0
            obuf[n, pl.ds(16, 16)] = (1.0 - a1 * idv) * mk1
            return c2
        lax.fori_loop(0, OB, _nd, 0)
        pltpu.sync_copy(obuf, out.at[oidx])
        return carry
    lax.fori_loop(0, NPT // OB, _st3, 0)


def kernel(t, y, edge_index, edge_attr, mask):
    del t
    src = edge_index[0]
    dst = edge_index[1]
    w = edge_attr
    ytr = y.T
    maskp = jnp.pad(mask.T, ((0, NPAD - N), (0, 0))).reshape(-1)
    zerosf = jnp.zeros((NPT * C,), jnp.float32)
    outp = _eikonal_sc(ytr, src, dst, w, maskp, zerosf)
    return outp[:N].T


# speedup vs baseline: 3.8099x; 1.1435x over previous
import functools

import jax
import jax.numpy as jnp
from jax import lax
from jax.experimental import pallas as pl
from jax.experimental.pallas import tpu as pltpu
from jax.experimental.pallas import tpu_sc as plsc

N = 100000
E = 1600000
C = 32
NC = 2
NS = 16
NT = NC * NS
NPT = 3136
NPAD = NT * NPT
K = 2000
G = 64
OB = 112
MEID = K + 96

_mesh = plsc.VectorSubcoreMesh(
    core_axis_name="c", subcore_axis_name="s", num_cores=NC, num_subcores=NS)


@functools.partial(
    pl.kernel,
    out_type=jax.ShapeDtypeStruct((NPAD, C), jnp.float32),
    mesh=_mesh,
    compiler_params=pltpu.CompilerParams(needs_layout_passes=False,
                                         use_tc_tiling_on_sc=False),
    scratch_types=[
        pltpu.VMEM((NPT * C,), jnp.float32),
        pltpu.VMEM((NPT + 16,), jnp.float32),
        pltpu.VMEM((K,), jnp.int32),
        pltpu.VMEM((K,), jnp.int32),
        pltpu.VMEM((K,), jnp.float32),
        pltpu.VMEM((MEID,), jnp.int32),
        pltpu.VMEM((MEID,), jnp.int32),
        pltpu.VMEM((MEID,), jnp.float32),
        pltpu.VMEM((G,), jnp.int32),
        pltpu.VMEM((G,), jnp.int32),
        pltpu.VMEM((G + 16,), jnp.float32),
        pltpu.VMEM((G + 16,), jnp.int32),
        pltpu.VMEM((G, C), jnp.float32),
        pltpu.VMEM((G, C), jnp.float32),
        pltpu.VMEM((OB * C,), jnp.float32),
        pltpu.VMEM((OB, C), jnp.float32),
        pltpu.VMEM((OB,), jnp.int32),
        pltpu.VMEM((64,), jnp.int32),
        pltpu.SemaphoreType.DMA,
        pltpu.SemaphoreType.DMA,
        pltpu.SemaphoreType.DMA,
    ],
)
def _eikonal_sc(ytr, srca, dsta, wa, maskp, zerosf, out,
                acc, deg, src_buf, dst_buf, w_buf, msrc, mdst, mw,
                sidx, didx, rbuf, slbuf, ysrc, ydst, mbuf, obuf, oidx,
                ptmp, sem1, sem2, sem3):
    cid = lax.axis_index("c")
    sid = lax.axis_index("s")
    wid = cid * NS + sid
    lo = wid * NPT
    hi = lo + NPT

    pltpu.sync_copy(zerosf, acc)
    pltpu.sync_copy(zerosf.at[pl.ds(0, NPT + 16)], deg)
    ptmp[pl.ds(0, 16)] = jnp.zeros((16,), jnp.int32)
    ptmp[pl.ds(32, 16)] = jnp.zeros((16,), jnp.int32)

    def _mz(k, carry):
        msrc[pl.ds(k * 16, 16)] = jnp.zeros((16,), jnp.int32)
        mdst[pl.ds(k * 16, 16)] = jnp.zeros((16,), jnp.int32)
        mw[pl.ds(k * 16, 16)] = jnp.full((16,), 1.0, jnp.float32)
        return carry
    lax.fori_loop(0, MEID // 16, _mz, 0)

    def _do_group(q0, cnt):
        def _prep(v, carry):
            iota = lax.iota(jnp.int32, 16)
            rows = iota + v * 16
            srcv = plsc.load_gather(msrc, [rows + q0])
            dstv = plsc.load_gather(mdst, [rows + q0])
            wv0 = plsc.load_gather(mw, [rows + q0])
            lane_ok = jnp.minimum(jnp.maximum(cnt - rows, 0),
                                  1).astype(jnp.float32)
            wv = wv0 * lane_ok + (1.0 - lane_ok)
            slv = srcv - lo
            slv = jnp.minimum(jnp.maximum(slv, 0), NPT - 1)
            wbits = plsc.bitcast(wv, jnp.int32)
            yb = 0x5F3759DF - lax.shift_right_logical(wbits, 1)
            yv = plsc.bitcast(yb, jnp.float32)
            hw = 0.5 * wv
            yv = yv * (1.5 - hw * yv * yv)
            yv = yv * (1.5 - hw * yv * yv)
            yv = yv * (1.5 - hw * yv * yv)
            sidx[pl.ds(v * 16, 16)] = srcv
            didx[pl.ds(v * 16, 16)] = dstv
            slbuf[pl.ds(v * 16, 16)] = slv
            rbuf[pl.ds(v * 16, 16)] = wv * yv
            plsc.addupdate_scatter(deg, [slv], wv * lane_ok)
            return carry
        lax.fori_loop(0, G // 16, _prep, 0)

        d1 = pltpu.async_copy(ytr.at[sidx], ysrc, sem1)
        d2 = pltpu.async_copy(ytr.at[didx], ydst, sem2)
        d1.wait()
        d2.wait()

        def _rmw(e, carry):
            iota = lax.iota(jnp.int32, 16)
            ef = jnp.full((16,), e, jnp.int32)
            sl = slbuf[pl.ds(e, 16)][0]
            rr = rbuf[pl.ds(e, 16)][0]
            ys0 = plsc.load_gather(ysrc, [ef, iota])
            ys1 = plsc.load_gather(ysrc, [ef, iota + 16])
            yd0 = plsc.load_gather(ydst, [ef, iota])
            yd1 = plsc.load_gather(ydst, [ef, iota + 16])
            v0 = rr * jnp.maximum(ys0 - yd0, 0.0)
            v1 = rr * jnp.maximum(ys1 - yd1, 0.0)
            o = sl * C
            a0 = acc[pl.ds(o, 16)]
            acc[pl.ds(o, 16)] = jnp.maximum(a0, v0)
            a1 = acc[pl.ds(o + 16, 16)]
            acc[pl.ds(o + 16, 16)] = jnp.maximum(a1, v1)
            return carry
        lax.fori_loop(0, cnt, _rmw, 0)

    def _chunk(i, cursor):
        off = i * K
        c1 = pltpu.async_copy(srca.at[pl.ds(off, K)], src_buf, sem1)
        c2 = pltpu.async_copy(dsta.at[pl.ds(off, K)], dst_buf, sem2)
        c3 = pltpu.async_copy(wa.at[pl.ds(off, K)], w_buf, sem3)
        c1.wait()
        c2.wait()
        c3.wait()

        def _scan(v, carry):
            sv = src_buf[pl.ds(v * 16, 16)]
            mi = jnp.minimum(jnp.maximum(jnp.minimum(sv - (lo - 1), hi - sv),
                                         0), 1)
            cs = mi
            for d in (1, 2, 4, 8):
                ptmp[pl.ds(16, 16)] = cs
                cs = cs + ptmp[pl.ds(16 - d, 16)]
            curv = ptmp[pl.ds(32, 16)]
            pos = (curv - 1) + cs
            posc = jnp.minimum(jnp.maximum(pos, 0), MEID - 2)
            posf = (MEID - 1) + mi * (posc - (MEID - 1))
            plsc.store_scatter(msrc, [posf], sv)
            plsc.store_scatter(mdst, [posf], dst_buf[pl.ds(v * 16, 16)])
            plsc.store_scatter(mw, [posf], w_buf[pl.ds(v * 16, 16)])
            ptmp[pl.ds(32, 16)] = curv + cs[15]
            return carry
        lax.fori_loop(0, K // 16, _scan, 0)
        cursor = ptmp[pl.ds(32, 16)][0]

        nfull = cursor // G

        def _grp(j, carry):
            _do_group(j * G, G)
            return carry
        lax.fori_loop(0, nfull, _grp, 0)

        rem = cursor - nfull * G
        base = nfull * G

        def _mv(k2, carry):
            msrc[pl.ds(k2 * 16, 16)] = msrc[pl.ds(base + k2 * 16, 16)]
            mdst[pl.ds(k2 * 16, 16)] = mdst[pl.ds(base + k2 * 16, 16)]
            mw[pl.ds(k2 * 16, 16)] = mw[pl.ds(base + k2 * 16, 16)]
            return carry
        lax.fori_loop(0, (rem + 15) // 16, _mv, 0)
        ptmp[pl.ds(32, 16)] = jnp.full((16,), 1, jnp.int32) * rem
        return rem
    cursor = lax.fori_loop(0, E // K, _chunk, 0)
    _do_group(0, cursor)

    def _inv(v, carry):
        d = deg[pl.ds(v * 16, 16)]
        deg[pl.ds(v * 16, 16)] = 1.0 / jnp.maximum(d, 1e-20)
        return carry
    lax.fori_loop(0, NPT // 16, _inv, 0)

    def _st3(ob, carry):
        nb = ob * OB
        pltpu.sync_copy(maskp.at[pl.ds((lo + nb) * C, OB * C)], mbuf)

        def _oi(v, c2):
            iota = lax.iota(jnp.int32, 16)
            oidx[pl.ds(v * 16, 16)] = iota + (lo + nb + v * 16)
            return c2
        lax.fori_loop(0, OB // 16, _oi, 0)

        def _nd(n, c2):
            idv = deg[pl.ds(nb + n, 16)][0]
            a0 = acc[pl.ds((nb + n) * C, 16)]
            mk0 = mbuf[pl.ds(n * C, 16)]
            a1 = acc[pl.ds((nb + n) * C + 16, 16)]
            mk1 = mbuf[pl.ds(n * C + 16, 16)]
            obuf[n, pl.ds(0, 16)] = (1.0 - a0 * idv) * mk0
            obuf[n, pl.ds(16, 16)] = (1.0 - a1 * idv) * mk1
            return c2
        lax.fori_loop(0, OB, _nd, 0)
        pltpu.sync_copy(obuf, out.at[oidx])
        return carry
    lax.fori_loop(0, NPT // OB, _st3, 0)


def kernel(t, y, edge_index, edge_attr, mask):
    del t
    src = edge_index[0]
    dst = edge_index[1]
    w = edge_attr
    ytr = y.T
    maskp = jnp.pad(mask.T, ((0, NPAD - N), (0, 0))).reshape(-1)
    zerosf = jnp.zeros((NPT * C,), jnp.float32)
    outp = _eikonal_sc(ytr, src, dst, w, maskp, zerosf)
    return outp[:N].T
